# baseline (speedup 1.0000x reference)
import functools

import jax
import jax.numpy as jnp
from jax import lax
from jax.experimental import pallas as pl
from jax.experimental.pallas import tpu as pltpu
from jax.experimental.pallas import tpu_sc as plsc

NUM_USERS = 100000
NUM_ITEMS = 100000
EMB_IN = 128
EMB_OUT = 300
BATCH = 16384

_NPIPE = 4
_BCHUNK = BATCH // _NPIPE

_NC = 2
_NS = 16
_NW = _NC * _NS
_BPW = _BCHUNK // _NW
_CHUNK = 128
_NSTREAM = _BPW // _CHUNK

_mesh = plsc.VectorSubcoreMesh(core_axis_name="c", subcore_axis_name="s")


@functools.partial(
    pl.kernel,
    mesh=_mesh,
    out_type=[
        jax.ShapeDtypeStruct((_BCHUNK, EMB_IN), jnp.float32),
        jax.ShapeDtypeStruct((_BCHUNK, EMB_IN), jnp.float32),
    ],
    scratch_types=[
        pltpu.VMEM((_NSTREAM, _CHUNK), jnp.int32),
        pltpu.VMEM((_NSTREAM, _CHUNK), jnp.int32),
        pltpu.VMEM((_BPW, EMB_IN), jnp.float32),
        pltpu.VMEM((_BPW, EMB_IN), jnp.float32),
        pltpu.SemaphoreType.DMA,
        pltpu.SemaphoreType.DMA,
    ],
)
def _sc_gather(ut_hbm, it_hbm, uidx_hbm, iidx_hbm, ue_out, ie_out,
               uidx_v, iidx_v, urows_v, irows_v, usem, isem):
    wid = lax.axis_index("s") * _NC + lax.axis_index("c")
    r0 = wid * _NSTREAM
    base = wid * _BPW
    pltpu.sync_copy(uidx_hbm.at[pl.ds(r0, _NSTREAM)], uidx_v)
    pltpu.sync_copy(iidx_hbm.at[pl.ds(r0, _NSTREAM)], iidx_v)
    ucopies = [
        pltpu.async_copy(ut_hbm.at[uidx_v.at[j]],
                         urows_v.at[pl.ds(j * _CHUNK, _CHUNK)], usem)
        for j in range(_NSTREAM)
    ]
    icopies = [
        pltpu.async_copy(it_hbm.at[iidx_v.at[j]],
                         irows_v.at[pl.ds(j * _CHUNK, _CHUNK)], isem)
        for j in range(_NSTREAM)
    ]
    for cp in ucopies:
        cp.wait()
    uout = pltpu.async_copy(urows_v, ue_out.at[pl.ds(base, _BPW)], usem)
    for cp in icopies:
        cp.wait()
    iout = pltpu.async_copy(irows_v, ie_out.at[pl.ds(base, _BPW)], isem)
    uout.wait()
    iout.wait()


_BLK = 1024
_NPAD = 384
_EPS = 1e-8


def _tc_body(ue_ref, ie_ref, w_ref, b_ref, out_ref):
    u = jnp.dot(ue_ref[...], w_ref[...], preferred_element_type=jnp.float32)
    v = jnp.dot(ie_ref[...], w_ref[...], preferred_element_type=jnp.float32)
    u = jnp.maximum(u + b_ref[...], 0.0)
    v = jnp.maximum(v + b_ref[...], 0.0)
    dot = jnp.sum(u * v, axis=1)
    nu = jnp.maximum(jnp.sqrt(jnp.sum(u * u, axis=1)), _EPS)
    nv = jnp.maximum(jnp.sqrt(jnp.sum(v * v, axis=1)), _EPS)
    out_ref[0, 0, :] = dot / (nu * nv)


_tc_call = pl.pallas_call(
    _tc_body,
    grid=(_BCHUNK // _BLK,),
    in_specs=[
        pl.BlockSpec((_BLK, EMB_IN), lambda i: (i, 0)),
        pl.BlockSpec((_BLK, EMB_IN), lambda i: (i, 0)),
        pl.BlockSpec((EMB_IN, _NPAD), lambda i: (0, 0)),
        pl.BlockSpec((1, _NPAD), lambda i: (0, 0)),
    ],
    out_specs=pl.BlockSpec((1, 1, _BLK), lambda i: (i, 0, 0)),
    out_shape=jax.ShapeDtypeStruct((_BCHUNK // _BLK, 1, _BLK), jnp.float32),
)


def kernel(user_indices, item_indices, user_table, item_table, W, b):
    uidx = user_indices.astype(jnp.int32).reshape(-1, _CHUNK)
    iidx = item_indices.astype(jnp.int32).reshape(-1, _CHUNK)
    w_pad = jnp.pad(W, ((0, 0), (0, _NPAD - EMB_OUT)))
    b_pad = jnp.pad(b, (0, _NPAD - EMB_OUT)).reshape(1, _NPAD)
    rows_per_chunk = _BCHUNK // _CHUNK
    scores = []
    for c in range(_NPIPE):
        lo, hi = c * rows_per_chunk, (c + 1) * rows_per_chunk
        ue, ie = _sc_gather(user_table, item_table, uidx[lo:hi], iidx[lo:hi])
        scores.append(_tc_call(ue, ie, w_pad, b_pad).reshape(_BCHUNK))
    return jnp.concatenate(scores)


# speedup vs baseline: 1.1986x; 1.1986x over previous
import functools

import jax
import jax.numpy as jnp
from jax import lax
from jax.experimental import pallas as pl
from jax.experimental.pallas import tpu as pltpu
from jax.experimental.pallas import tpu_sc as plsc

NUM_USERS = 100000
NUM_ITEMS = 100000
EMB_IN = 128
EMB_OUT = 300
BATCH = 16384

_NPIPE = 2
_BCHUNK = BATCH // _NPIPE

_NC = 2
_NS = 16
_NW = _NC * _NS
_BPW = _BCHUNK // _NW
_CHUNK = 128
_NSTREAM = _BPW // _CHUNK

_mesh = plsc.VectorSubcoreMesh(core_axis_name="c", subcore_axis_name="s")


@functools.partial(
    pl.kernel,
    mesh=_mesh,
    out_type=[
        jax.ShapeDtypeStruct((_BCHUNK, EMB_IN), jnp.float32),
        jax.ShapeDtypeStruct((_BCHUNK, EMB_IN), jnp.float32),
    ],
    scratch_types=[
        pltpu.VMEM((_NSTREAM, _CHUNK), jnp.int32),
        pltpu.VMEM((_NSTREAM, _CHUNK), jnp.int32),
        pltpu.VMEM((_BPW, EMB_IN), jnp.float32),
        pltpu.VMEM((_BPW, EMB_IN), jnp.float32),
        pltpu.SemaphoreType.DMA,
        pltpu.SemaphoreType.DMA,
    ],
)
def _sc_gather(ut_hbm, it_hbm, uidx_hbm, iidx_hbm, ue_out, ie_out,
               uidx_v, iidx_v, urows_v, irows_v, usem, isem):
    wid = lax.axis_index("s") * _NC + lax.axis_index("c")
    r0 = wid * _NSTREAM
    base = wid * _BPW
    pltpu.sync_copy(uidx_hbm.at[pl.ds(r0, _NSTREAM)], uidx_v)
    pltpu.sync_copy(iidx_hbm.at[pl.ds(r0, _NSTREAM)], iidx_v)
    ucopies = [
        pltpu.async_copy(ut_hbm.at[uidx_v.at[j]],
                         urows_v.at[pl.ds(j * _CHUNK, _CHUNK)], usem)
        for j in range(_NSTREAM)
    ]
    icopies = [
        pltpu.async_copy(it_hbm.at[iidx_v.at[j]],
                         irows_v.at[pl.ds(j * _CHUNK, _CHUNK)], isem)
        for j in range(_NSTREAM)
    ]
    for cp in ucopies:
        cp.wait()
    uout = pltpu.async_copy(urows_v, ue_out.at[pl.ds(base, _BPW)], usem)
    for cp in icopies:
        cp.wait()
    iout = pltpu.async_copy(irows_v, ie_out.at[pl.ds(base, _BPW)], isem)
    uout.wait()
    iout.wait()


_BLK = 1024
_NPAD = 384
_EPS = 1e-8


_NT = (((1,), (1,)), ((), ()))


def _tc_body(ue_ref, ie_ref, wt_ref, bt_ref, out_ref):
    ut = jax.lax.dot_general(wt_ref[...], ue_ref[...], _NT,
                             preferred_element_type=jnp.float32)
    vt = jax.lax.dot_general(wt_ref[...], ie_ref[...], _NT,
                             preferred_element_type=jnp.float32)
    bt = bt_ref[...]
    ut = jnp.maximum(ut + bt, 0.0)
    vt = jnp.maximum(vt + bt, 0.0)
    dot = jnp.sum(ut * vt, axis=0)
    nu2 = jnp.sum(ut * ut, axis=0)
    nv2 = jnp.sum(vt * vt, axis=0)
    denom2 = jnp.maximum(nu2, _EPS * _EPS) * jnp.maximum(nv2, _EPS * _EPS)
    out_ref[0, 0, :] = dot * jax.lax.rsqrt(denom2)


_tc_call = pl.pallas_call(
    _tc_body,
    grid=(_BCHUNK // _BLK,),
    in_specs=[
        pl.BlockSpec((_BLK, EMB_IN), lambda i: (i, 0)),
        pl.BlockSpec((_BLK, EMB_IN), lambda i: (i, 0)),
        pl.BlockSpec((_NPAD, EMB_IN), lambda i: (0, 0)),
        pl.BlockSpec((_NPAD, 1), lambda i: (0, 0)),
    ],
    out_specs=pl.BlockSpec((1, 1, _BLK), lambda i: (i, 0, 0)),
    out_shape=jax.ShapeDtypeStruct((_BCHUNK // _BLK, 1, _BLK), jnp.float32),
)


def kernel(user_indices, item_indices, user_table, item_table, W, b):
    uidx = user_indices.astype(jnp.int32).reshape(-1, _CHUNK)
    iidx = item_indices.astype(jnp.int32).reshape(-1, _CHUNK)
    w_pad = jnp.pad(W, ((0, 0), (0, _NPAD - EMB_OUT))).T
    b_pad = jnp.pad(b, (0, _NPAD - EMB_OUT)).reshape(_NPAD, 1)
    rows_per_chunk = _BCHUNK // _CHUNK
    scores = []
    for c in range(_NPIPE):
        lo, hi = c * rows_per_chunk, (c + 1) * rows_per_chunk
        ue, ie = _sc_gather(user_table, item_table, uidx[lo:hi], iidx[lo:hi])
        scores.append(_tc_call(ue, ie, w_pad, b_pad).reshape(_BCHUNK))
    return jnp.concatenate(scores)
